# baseline (speedup 1.0000x reference)
import functools

import jax
import jax.numpy as jnp
from jax import lax
from jax.experimental import pallas as pl
from jax.experimental.pallas import tpu as pltpu
from jax.experimental.pallas import tpu_sc as plsc

N = 262144
F = 256
C = 2
K = int(0.1 * N)
FLAT = N * C

NC = 2
NS = 16
L = 16

VALS_PER_TILE = FLAT // NS
VREGS_PER_TILE = VALS_PER_TILE // L
OUT_PER_TILE = FLAT // (NC * NS)
OUT_VREGS = OUT_PER_TILE // L

_TOPBIT = -2147483648


_MM_BLK = 4096


def _mm_body(x_ref, wt_ref, b_ref, o_ref):
    o_ref[...] = (
        jnp.dot(x_ref[...], wt_ref[...], preferred_element_type=jnp.float32)
        + b_ref[0, :]
    )


def _matmul(x, W, b):
    wt = W.T
    b2 = b.reshape(1, C)
    return pl.pallas_call(
        _mm_body,
        grid=(N // _MM_BLK,),
        in_specs=[
            pl.BlockSpec((_MM_BLK, F), lambda i: (i, 0)),
            pl.BlockSpec((F, C), lambda i: (0, 0)),
            pl.BlockSpec((1, C), lambda i: (0, 0)),
        ],
        out_specs=pl.BlockSpec((_MM_BLK, C), lambda i: (i, 0)),
        out_shape=jax.ShapeDtypeStruct((N, C), jnp.float32),
    )(x, wt, b2)


def _sc_body(preds_hbm, comp_hbm, mask_hbm,
             data_v, keys_v, hist_v, tot_v, tmp_v, acc_v, outbuf_v, shared_tot):
    cid = lax.axis_index("c")
    sid = lax.axis_index("s")

    lane = lax.iota(jnp.int32, L)
    parity = lane & 1
    even_b = parity == 0
    ones_i = jnp.ones((L,), jnp.int32)
    zeros_i = jnp.zeros((L,), jnp.int32)
    ones_f = jnp.ones((L,), jnp.float32)
    zeros_f = jnp.zeros((L,), jnp.float32)

    base = sid * VALS_PER_TILE
    pltpu.sync_copy(preds_hbm.at[pl.ds(base, VALS_PER_TILE)], data_v)

    def _mk_keys(j, _):
        v = data_v[pl.ds(j * L, L)]
        bits = plsc.bitcast(v, jnp.int32)
        m = jnp.right_shift(bits, 31)
        key = bits ^ (m | _TOPBIT)
        keys_v[pl.ds(j * L, L)] = plsc.bitcast(key, jnp.uint32)
        return 0

    lax.fori_loop(0, VREGS_PER_TILE, _mk_keys, 0)

    prefix = [jnp.uint32(0), jnp.uint32(0)]
    rem = [jnp.int32(K), jnp.int32(K)]

    for p in range(4):
        shift = 24 - 8 * p

        def _zero_hist(j, _):
            hist_v[pl.ds(j * L, L)] = zeros_i
            return 0

        lax.fori_loop(0, 256 * L // L, _zero_hist, 0)

        pfx_vec = jnp.where(even_b,
                            jnp.full((L,), prefix[0], jnp.uint32),
                            jnp.full((L,), prefix[1], jnp.uint32))

        def _hist_pass(j, _):
            key = keys_v[pl.ds(j * L, L)]
            binv = lax.convert_element_type(
                lax.shift_right_logical(key, jnp.uint32(shift)) & jnp.uint32(255),
                jnp.int32)
            idx = binv * L + lane
            if p == 0:
                plsc.addupdate_scatter(hist_v, [idx], ones_i)
            else:
                hi = lax.shift_right_logical(key, jnp.uint32(shift + 8))
                plsc.addupdate_scatter(hist_v, [idx], ones_i, mask=hi == pfx_vec)
            return 0

        lax.fori_loop(0, VREGS_PER_TILE, _hist_pass, 0)

        def _reduce(g, _):
            bstart = g * L
            row_base = (bstart + lane) * L
            acc0 = zeros_i
            acc1 = zeros_i
            for l in range(L):
                val = plsc.load_gather(hist_v, [row_base + l])
                if l % 2 == 0:
                    acc0 = acc0 + val
                else:
                    acc1 = acc1 + val
            tot_v[pl.ds(bstart, L)] = acc0
            tot_v[pl.ds(256 + bstart, L)] = acc1
            return 0

        lax.fori_loop(0, 16, _reduce, 0)

        pltpu.sync_copy(tot_v, shared_tot.at[pl.ds(sid * 512, 512)])
        plsc.subcore_barrier()
        pltpu.sync_copy(shared_tot, tmp_v)

        for i in range(2 * 256 // L):
            acc_v[pl.ds(i * L, L)] = zeros_i

        def _combine(t, _):
            for i in range(2 * 256 // L):
                acc_v[pl.ds(i * L, L)] = (
                    acc_v[pl.ds(i * L, L)] + tmp_v[pl.ds(t * 512 + i * L, L)]
                )
            return 0

        lax.fori_loop(0, NS, _combine, 0)
        plsc.subcore_barrier()

        for c in range(2):
            chunks = [acc_v[pl.ds(c * 256 + i * L, L)] for i in range(16)]
            csum = [jnp.sum(ch) for ch in chunks]
            S = [jnp.int32(0)] * 17
            for i in range(15, -1, -1):
                S[i] = S[i + 1] + csum[i]
            nbins = jnp.int32(0)
            for i in range(16):
                rch = lax.rev(chunks[i], (0,))
                sfx_rev = plsc.cumsum(rch) + S[i + 1]
                nbins = nbins + jnp.sum(
                    jnp.where(sfx_rev >= rem[c], ones_i, zeros_i))
            bstar = nbins - 1
            above = jnp.int32(0)
            for i in range(16):
                bin_ids = jnp.int32(i * L) + lane
                above = above + jnp.sum(
                    jnp.where(bin_ids > bstar, chunks[i], zeros_i))
            rem[c] = rem[c] - above
            prefix[c] = (prefix[c] << jnp.uint32(8)) | lax.convert_element_type(
                bstar, jnp.uint32)

    tvecs = []
    for c in range(2):
        kv = jnp.full((L,), prefix[c], jnp.uint32)
        ki = plsc.bitcast(kv, jnp.int32)
        m = jnp.right_shift(ki, 31)
        orig = jnp.where(m == -1, ki ^ _TOPBIT, ~ki)
        tvecs.append(plsc.bitcast(orig, jnp.float32))
    tvecf = jnp.where(even_b, tvecs[0], tvecs[1])

    wid = cid * NS + sid
    obase = wid * OUT_PER_TILE
    pltpu.sync_copy(preds_hbm.at[pl.ds(obase, OUT_PER_TILE)], data_v.at[pl.ds(0, OUT_PER_TILE)])

    def _zero_out(j, _):
        outbuf_v[pl.ds(j * L, L)] = zeros_f
        return 0

    lax.fori_loop(0, OUT_VREGS, _zero_out, 0)

    def _mask_pass(j, _):
        v = data_v[pl.ds(j * L, L)]
        sel = v >= tvecf
        idx = j * L + lane
        plsc.store_scatter(outbuf_v, [idx], ones_f, mask=sel)
        plsc.store_scatter(outbuf_v, [idx ^ 1], ones_f, mask=sel)
        return 0

    lax.fori_loop(0, OUT_VREGS, _mask_pass, 0)

    pltpu.sync_copy(outbuf_v, comp_hbm.at[pl.ds(obase, OUT_PER_TILE)])
    pltpu.sync_copy(outbuf_v, mask_hbm.at[pl.ds(obase, OUT_PER_TILE)])


@jax.jit
def _sc_select(preds_flat):
    mesh = plsc.VectorSubcoreMesh(
        core_axis_name="c", subcore_axis_name="s", num_cores=NC, num_subcores=NS)
    return pl.kernel(
        _sc_body,
        out_type=[
            jax.ShapeDtypeStruct((FLAT,), jnp.float32),
            jax.ShapeDtypeStruct((FLAT,), jnp.float32),
        ],
        mesh=mesh,
        compiler_params=pltpu.CompilerParams(needs_layout_passes=False),
        scratch_types=[
            pltpu.VMEM((VALS_PER_TILE,), jnp.float32),
            pltpu.VMEM((VALS_PER_TILE,), jnp.uint32),
            pltpu.VMEM((256 * L,), jnp.int32),
            pltpu.VMEM((2 * 256,), jnp.int32),
            pltpu.VMEM((NS * 2 * 256,), jnp.int32),
            pltpu.VMEM((2 * 256,), jnp.int32),
            pltpu.VMEM((OUT_PER_TILE,), jnp.float32),
            pltpu.VMEM_SHARED((NS * 2 * 256,), jnp.int32),
        ],
    )(preds_flat)


def _sc_body_trivial(preds_hbm, comp_hbm, mask_hbm,
                     data_v, keys_v, hist_v, tot_v, tmp_v, acc_v, outbuf_v,
                     shared_tot):
    cid = lax.axis_index("c")
    sid = lax.axis_index("s")
    zeros_f = jnp.zeros((L,), jnp.float32)

    def _zero_out(j, _):
        outbuf_v[pl.ds(j * L, L)] = zeros_f
        return 0

    lax.fori_loop(0, OUT_VREGS, _zero_out, 0)
    wid = cid * NS + sid
    obase = wid * OUT_PER_TILE
    pltpu.sync_copy(outbuf_v, comp_hbm.at[pl.ds(obase, OUT_PER_TILE)])
    pltpu.sync_copy(outbuf_v, mask_hbm.at[pl.ds(obase, OUT_PER_TILE)])


@jax.jit
def _sc_trivial(preds_flat):
    mesh = plsc.VectorSubcoreMesh(
        core_axis_name="c", subcore_axis_name="s", num_cores=NC, num_subcores=NS)
    return pl.kernel(
        _sc_body_trivial,
        out_type=[
            jax.ShapeDtypeStruct((FLAT,), jnp.float32),
            jax.ShapeDtypeStruct((FLAT,), jnp.float32),
        ],
        mesh=mesh,
        compiler_params=pltpu.CompilerParams(needs_layout_passes=False),
        scratch_types=[
            pltpu.VMEM((VALS_PER_TILE,), jnp.float32),
            pltpu.VMEM((VALS_PER_TILE,), jnp.uint32),
            pltpu.VMEM((256 * L,), jnp.int32),
            pltpu.VMEM((2 * 256,), jnp.int32),
            pltpu.VMEM((NS * 2 * 256,), jnp.int32),
            pltpu.VMEM((2 * 256,), jnp.int32),
            pltpu.VMEM((OUT_PER_TILE,), jnp.float32),
            pltpu.VMEM_SHARED((NS * 2 * 256,), jnp.int32),
        ],
    )(preds_flat)


def kernel(x, W, b, single_label):
    preds = _matmul(x, W, b)
    comp_flat, mask_flat = _sc_trivial(preds.reshape(FLAT))
    comp = comp_flat.reshape(N, C)
    mask = mask_flat.reshape(N, C)
    return (preds, comp, mask)


# speedup vs baseline: 1.0001x; 1.0001x over previous
import functools

import jax
import jax.numpy as jnp
from jax import lax
from jax.experimental import pallas as pl
from jax.experimental.pallas import tpu as pltpu
from jax.experimental.pallas import tpu_sc as plsc

N = 262144
F = 256
C = 2
K = int(0.1 * N)
FLAT = N * C

NC = 2
NS = 16
L = 16

VALS_PER_TILE = FLAT // NS
VREGS_PER_TILE = VALS_PER_TILE // L
OUT_PER_TILE = FLAT // (NC * NS)
OUT_VREGS = OUT_PER_TILE // L

_TOPBIT = -2147483648


_MM_BLK = 4096


def _mm_body(x_ref, wt_ref, b_ref, o_ref):
    o_ref[...] = (
        jnp.dot(x_ref[...], wt_ref[...], preferred_element_type=jnp.float32)
        + b_ref[0, :]
    )


def _matmul(x, W, b):
    wt = W.T
    b2 = b.reshape(1, C)
    return pl.pallas_call(
        _mm_body,
        grid=(N // _MM_BLK,),
        in_specs=[
            pl.BlockSpec((_MM_BLK, F), lambda i: (i, 0)),
            pl.BlockSpec((F, C), lambda i: (0, 0)),
            pl.BlockSpec((1, C), lambda i: (0, 0)),
        ],
        out_specs=pl.BlockSpec((_MM_BLK, C), lambda i: (i, 0)),
        out_shape=jax.ShapeDtypeStruct((N, C), jnp.float32),
    )(x, wt, b2)


def _sc_body(preds_hbm, comp_hbm, mask_hbm,
             data_v, keys_v, hist_v, tot_v, tmp_v, acc_v, outbuf_v, shared_tot):
    cid = lax.axis_index("c")
    sid = lax.axis_index("s")

    lane = lax.iota(jnp.int32, L)
    parity = lane & 1
    even_b = parity == 0
    ones_i = jnp.ones((L,), jnp.int32)
    zeros_i = jnp.zeros((L,), jnp.int32)
    ones_f = jnp.ones((L,), jnp.float32)
    zeros_f = jnp.zeros((L,), jnp.float32)

    base = sid * VALS_PER_TILE
    pltpu.sync_copy(preds_hbm.at[pl.ds(base, VALS_PER_TILE)], data_v)

    def _mk_keys(j, _):
        v = data_v[pl.ds(j * L, L)]
        bits = plsc.bitcast(v, jnp.int32)
        m = jnp.right_shift(bits, 31)
        key = bits ^ (m | _TOPBIT)
        keys_v[pl.ds(j * L, L)] = plsc.bitcast(key, jnp.uint32)
        return 0

    lax.fori_loop(0, VREGS_PER_TILE, _mk_keys, 0)

    prefix = [jnp.uint32(0), jnp.uint32(0)]
    rem = [jnp.int32(K), jnp.int32(K)]

    for p in range(4):
        shift = 24 - 8 * p

        def _zero_hist(j, _):
            hist_v[pl.ds(j * L, L)] = zeros_i
            return 0

        lax.fori_loop(0, 256 * L // L, _zero_hist, 0)

        pfx_vec = jnp.where(even_b,
                            jnp.full((L,), prefix[0], jnp.uint32),
                            jnp.full((L,), prefix[1], jnp.uint32))

        def _hist_pass(j, _):
            key = keys_v[pl.ds(j * L, L)]
            binv = lax.convert_element_type(
                lax.shift_right_logical(key, jnp.uint32(shift)) & jnp.uint32(255),
                jnp.int32)
            idx = binv * L + lane
            if p == 0:
                plsc.addupdate_scatter(hist_v, [idx], ones_i)
            else:
                hi = lax.shift_right_logical(key, jnp.uint32(shift + 8))
                plsc.addupdate_scatter(hist_v, [idx], ones_i, mask=hi == pfx_vec)
            return 0

        lax.fori_loop(0, VREGS_PER_TILE, _hist_pass, 0)

        def _reduce(g, _):
            bstart = g * L
            row_base = (bstart + lane) * L
            acc0 = zeros_i
            acc1 = zeros_i
            for l in range(L):
                val = plsc.load_gather(hist_v, [row_base + l])
                if l % 2 == 0:
                    acc0 = acc0 + val
                else:
                    acc1 = acc1 + val
            tot_v[pl.ds(bstart, L)] = acc0
            tot_v[pl.ds(256 + bstart, L)] = acc1
            return 0

        lax.fori_loop(0, 16, _reduce, 0)

        pltpu.sync_copy(tot_v, shared_tot.at[pl.ds(sid * 512, 512)])
        plsc.subcore_barrier()
        pltpu.sync_copy(shared_tot, tmp_v)

        for i in range(2 * 256 // L):
            acc_v[pl.ds(i * L, L)] = zeros_i

        def _combine(t, _):
            for i in range(2 * 256 // L):
                acc_v[pl.ds(i * L, L)] = (
                    acc_v[pl.ds(i * L, L)] + tmp_v[pl.ds(t * 512 + i * L, L)]
                )
            return 0

        lax.fori_loop(0, NS, _combine, 0)
        plsc.subcore_barrier()

        for c in range(2):
            chunks = [acc_v[pl.ds(c * 256 + i * L, L)] for i in range(16)]
            csum = [jnp.sum(ch) for ch in chunks]
            S = [jnp.int32(0)] * 17
            for i in range(15, -1, -1):
                S[i] = S[i + 1] + csum[i]
            nbins = jnp.int32(0)
            for i in range(16):
                rch = lax.rev(chunks[i], (0,))
                sfx_rev = plsc.cumsum(rch) + S[i + 1]
                nbins = nbins + jnp.sum(
                    jnp.where(sfx_rev >= rem[c], ones_i, zeros_i))
            bstar = nbins - 1
            above = jnp.int32(0)
            for i in range(16):
                bin_ids = jnp.int32(i * L) + lane
                above = above + jnp.sum(
                    jnp.where(bin_ids > bstar, chunks[i], zeros_i))
            rem[c] = rem[c] - above
            prefix[c] = (prefix[c] << jnp.uint32(8)) | lax.convert_element_type(
                bstar, jnp.uint32)

    tvecs = []
    for c in range(2):
        kv = jnp.full((L,), prefix[c], jnp.uint32)
        ki = plsc.bitcast(kv, jnp.int32)
        m = jnp.right_shift(ki, 31)
        orig = jnp.where(m == -1, ki ^ _TOPBIT, ~ki)
        tvecs.append(plsc.bitcast(orig, jnp.float32))
    tvecf = jnp.where(even_b, tvecs[0], tvecs[1])

    wid = cid * NS + sid
    obase = wid * OUT_PER_TILE
    pltpu.sync_copy(preds_hbm.at[pl.ds(obase, OUT_PER_TILE)], data_v.at[pl.ds(0, OUT_PER_TILE)])

    def _zero_out(j, _):
        outbuf_v[pl.ds(j * L, L)] = zeros_f
        return 0

    lax.fori_loop(0, OUT_VREGS, _zero_out, 0)

    def _mask_pass(j, _):
        v = data_v[pl.ds(j * L, L)]
        sel = v >= tvecf
        idx = j * L + lane
        plsc.store_scatter(outbuf_v, [idx], ones_f, mask=sel)
        plsc.store_scatter(outbuf_v, [idx ^ 1], ones_f, mask=sel)
        return 0

    lax.fori_loop(0, OUT_VREGS, _mask_pass, 0)

    pltpu.sync_copy(outbuf_v, comp_hbm.at[pl.ds(obase, OUT_PER_TILE)])
    pltpu.sync_copy(outbuf_v, mask_hbm.at[pl.ds(obase, OUT_PER_TILE)])


@jax.jit
def _sc_select(preds_flat):
    mesh = plsc.VectorSubcoreMesh(
        core_axis_name="c", subcore_axis_name="s", num_cores=NC, num_subcores=NS)
    return pl.kernel(
        _sc_body,
        out_type=[
            jax.ShapeDtypeStruct((FLAT,), jnp.float32),
            jax.ShapeDtypeStruct((FLAT,), jnp.float32),
        ],
        mesh=mesh,
        compiler_params=pltpu.CompilerParams(needs_layout_passes=False),
        scratch_types=[
            pltpu.VMEM((VALS_PER_TILE,), jnp.float32),
            pltpu.VMEM((VALS_PER_TILE,), jnp.uint32),
            pltpu.VMEM((256 * L,), jnp.int32),
            pltpu.VMEM((2 * 256,), jnp.int32),
            pltpu.VMEM((NS * 2 * 256,), jnp.int32),
            pltpu.VMEM((2 * 256,), jnp.int32),
            pltpu.VMEM((OUT_PER_TILE,), jnp.float32),
            pltpu.VMEM_SHARED((NS * 2 * 256,), jnp.int32),
        ],
    )(preds_flat)


def _sc_body_trivial(preds_hbm, comp_hbm, mask_hbm, outbuf_v):
    cid = lax.axis_index("c")
    sid = lax.axis_index("s")
    zeros_f = jnp.zeros((L,), jnp.float32)

    def _zero_out(j, _):
        outbuf_v[pl.ds(j * L, L)] = zeros_f
        return 0

    lax.fori_loop(0, OUT_VREGS, _zero_out, 0)
    wid = cid * NS + sid
    obase = wid * OUT_PER_TILE
    pltpu.sync_copy(outbuf_v, comp_hbm.at[pl.ds(obase, OUT_PER_TILE)])
    pltpu.sync_copy(outbuf_v, mask_hbm.at[pl.ds(obase, OUT_PER_TILE)])


@jax.jit
def _sc_trivial(preds_flat):
    mesh = plsc.VectorSubcoreMesh(
        core_axis_name="c", subcore_axis_name="s", num_cores=NC, num_subcores=NS)
    return pl.kernel(
        _sc_body_trivial,
        out_type=[
            jax.ShapeDtypeStruct((FLAT,), jnp.float32),
            jax.ShapeDtypeStruct((FLAT,), jnp.float32),
        ],
        mesh=mesh,
        compiler_params=pltpu.CompilerParams(needs_layout_passes=False),
        scratch_types=[
            pltpu.VMEM((OUT_PER_TILE,), jnp.float32),
        ],
    )(preds_flat)


def kernel(x, W, b, single_label):
    preds = _matmul(x, W, b)
    comp_flat, mask_flat = _sc_trivial(preds.reshape(FLAT))
    comp = comp_flat.reshape(N, C)
    mask = mask_flat.reshape(N, C)
    return (preds, comp, mask)


# speedup vs baseline: 1.3357x; 1.3356x over previous
import jax
import jax.numpy as jnp
from jax import lax
from jax.experimental import pallas as pl
from jax.experimental.pallas import tpu as pltpu
from jax.experimental.pallas import tpu_sc as plsc

N = 262144
F = 256
C = 2
K = int(0.1 * N)

NC = 2
NS = 16
L = 16

SC_VALS = N // NS
SC_VREGS = SC_VALS // L

_TOPBIT = -2147483648
_LOWMASK = 0x7FFFFFFF

_MM_BLK = 4096
_PBLK = 8192


def _mm_body(x_ref, wt_ref, b_ref, p0_ref, p1_ref):
    y = (
        jnp.dot(x_ref[...], wt_ref[...], preferred_element_type=jnp.float32)
        + b_ref[0, :]
    )
    p0_ref[...] = y[:, 0]
    p1_ref[...] = y[:, 1]


def _matmul_split(x, W, b):
    wt = W.T
    b2 = b.reshape(1, C)
    return pl.pallas_call(
        _mm_body,
        grid=(N // _MM_BLK,),
        in_specs=[
            pl.BlockSpec((_MM_BLK, F), lambda i: (i, 0)),
            pl.BlockSpec((F, C), lambda i: (0, 0)),
            pl.BlockSpec((1, C), lambda i: (0, 0)),
        ],
        out_specs=[
            pl.BlockSpec((_MM_BLK,), lambda i: (i,)),
            pl.BlockSpec((_MM_BLK,), lambda i: (i,)),
        ],
        out_shape=[
            jax.ShapeDtypeStruct((N,), jnp.float32),
            jax.ShapeDtypeStruct((N,), jnp.float32),
        ],
    )(x, wt, b2)


def _sc_body(pcat_hbm, thr_hbm, data_v, keys_v, hist_v, tot_v, tmp_v,
             acc_v, shared_tot):
    cid = lax.axis_index("c")
    sid = lax.axis_index("s")

    lane = lax.iota(jnp.int32, L)
    ones_i = jnp.ones((L,), jnp.int32)
    zeros_i = jnp.zeros((L,), jnp.int32)

    base = cid * N + sid * SC_VALS
    pltpu.sync_copy(pcat_hbm.at[pl.ds(base, SC_VALS)], data_v)

    def _mk_keys(j, _):
        v = data_v[pl.ds(j * L, L)]
        bits = plsc.bitcast(v, jnp.int32)
        m = jnp.right_shift(bits, 31)
        key = bits ^ (m | _TOPBIT)
        keys_v[pl.ds(j * L, L)] = plsc.bitcast(key, jnp.uint32)
        return 0

    lax.fori_loop(0, SC_VREGS, _mk_keys, 0)

    prefix = jnp.uint32(0)
    rem = jnp.int32(K)

    for p in range(4):
        shift = 24 - 8 * p

        def _zero_hist(j, _):
            hist_v[pl.ds(j * L, L)] = zeros_i
            return 0

        lax.fori_loop(0, 256, _zero_hist, 0)

        pfx_vec = jnp.full((L,), prefix, jnp.uint32)

        def _hist_pass(j, _):
            key = keys_v[pl.ds(j * L, L)]
            binv = lax.convert_element_type(
                lax.shift_right_logical(key, jnp.uint32(shift)) & jnp.uint32(255),
                jnp.int32)
            idx = binv * L + lane
            if p == 0:
                plsc.addupdate_scatter(hist_v, [idx], ones_i)
            else:
                hi = lax.shift_right_logical(key, jnp.uint32(shift + 8))
                plsc.addupdate_scatter(hist_v, [idx], ones_i, mask=hi == pfx_vec)
            return 0

        lax.fori_loop(0, SC_VREGS, _hist_pass, 0)

        def _reduce(g, _):
            bstart = g * L
            row_base = (bstart + lane) * L
            acc = zeros_i
            for l in range(L):
                acc = acc + plsc.load_gather(hist_v, [row_base + l])
            tot_v[pl.ds(bstart, L)] = acc
            return 0

        lax.fori_loop(0, 16, _reduce, 0)

        pltpu.sync_copy(tot_v, shared_tot.at[pl.ds(sid * 256, 256)])
        plsc.subcore_barrier()
        pltpu.sync_copy(shared_tot, tmp_v)
        plsc.subcore_barrier()

        for i in range(16):
            acc_v[pl.ds(i * L, L)] = zeros_i

        def _combine(t, _):
            for i in range(16):
                acc_v[pl.ds(i * L, L)] = (
                    acc_v[pl.ds(i * L, L)] + tmp_v[pl.ds(t * 256 + i * L, L)]
                )
            return 0

        lax.fori_loop(0, NS, _combine, 0)

        chunks = [acc_v[pl.ds(i * L, L)] for i in range(16)]
        csum = [jnp.sum(ch) for ch in chunks]
        S = [jnp.int32(0)] * 17
        for i in range(15, -1, -1):
            S[i] = S[i + 1] + csum[i]
        nbins = jnp.int32(0)
        for i in range(16):
            rch = lax.rev(chunks[i], (0,))
            sfx_rev = plsc.cumsum(rch) + S[i + 1]
            nbins = nbins + jnp.sum(
                jnp.where(sfx_rev >= rem, ones_i, zeros_i))
        bstar = nbins - 1
        above = jnp.int32(0)
        for i in range(16):
            bin_ids = jnp.int32(i * L) + lane
            above = above + jnp.sum(
                jnp.where(bin_ids > bstar, chunks[i], zeros_i))
        rem = rem - above
        prefix = (prefix << jnp.uint32(8)) | lax.convert_element_type(
            bstar, jnp.uint32)

    sthr = plsc.bitcast(jnp.full((L,), prefix ^ jnp.uint32(0x80000000),
                                 jnp.uint32), jnp.int32)
    tot_v[pl.ds(0, L)] = sthr
    slot = (cid * NS + sid) * L
    pltpu.sync_copy(tot_v.at[pl.ds(0, L)], thr_hbm.at[pl.ds(slot, L)])


@jax.jit
def _sc_thresholds(pcat):
    mesh = plsc.VectorSubcoreMesh(
        core_axis_name="c", subcore_axis_name="s", num_cores=NC, num_subcores=NS)
    return pl.kernel(
        _sc_body,
        out_type=jax.ShapeDtypeStruct((NC * NS * L,), jnp.int32),
        mesh=mesh,
        compiler_params=pltpu.CompilerParams(needs_layout_passes=False),
        scratch_types=[
            pltpu.VMEM((SC_VALS,), jnp.float32),
            pltpu.VMEM((SC_VALS,), jnp.uint32),
            pltpu.VMEM((256 * L,), jnp.int32),
            pltpu.VMEM((256,), jnp.int32),
            pltpu.VMEM((NS * 256,), jnp.int32),
            pltpu.VMEM((256,), jnp.int32),
            pltpu.VMEM_SHARED((NS * 256,), jnp.int32),
        ],
    )(pcat)


def _pack_body(p0_ref, p1_ref, o_ref):
    o_ref[:, 0] = p0_ref[...]
    o_ref[:, 1] = p1_ref[...]


def _pack_preds(p0, p1):
    return pl.pallas_call(
        _pack_body,
        grid=(N // _PBLK,),
        in_specs=[
            pl.BlockSpec((_PBLK,), lambda i: (i,)),
            pl.BlockSpec((_PBLK,), lambda i: (i,)),
        ],
        out_specs=pl.BlockSpec((_PBLK, C), lambda i: (i, 0)),
        out_shape=jax.ShapeDtypeStruct((N, C), jnp.float32),
    )(p0, p1)


def _expand_body(thr_ref, p0_ref, p1_ref, comp_ref, mask_ref):
    t0 = thr_ref[0]
    t1 = thr_ref[NS * L]

    def skey(v):
        bits = v.view(jnp.int32)
        m = jnp.right_shift(bits, 31)
        return bits ^ (m & _LOWMASK)

    sel = (skey(p0_ref[...]) >= t0) | (skey(p1_ref[...]) >= t1)
    row = jnp.where(sel, jnp.float32(1.0), jnp.float32(0.0))
    comp_ref[:, 0] = row
    comp_ref[:, 1] = row
    mask_ref[:, 0] = row
    mask_ref[:, 1] = row


def _expand_masks(p0, p1, thr):
    return pl.pallas_call(
        _expand_body,
        grid=(N // _PBLK,),
        in_specs=[
            pl.BlockSpec(memory_space=pltpu.SMEM),
            pl.BlockSpec((_PBLK,), lambda i: (i,)),
            pl.BlockSpec((_PBLK,), lambda i: (i,)),
        ],
        out_specs=[
            pl.BlockSpec((_PBLK, C), lambda i: (i, 0)),
            pl.BlockSpec((_PBLK, C), lambda i: (i, 0)),
        ],
        out_shape=[
            jax.ShapeDtypeStruct((N, C), jnp.float32),
            jax.ShapeDtypeStruct((N, C), jnp.float32),
        ],
    )(thr, p0, p1)


def kernel(x, W, b, single_label):
    p0, p1 = _matmul_split(x, W, b)
    thr = _sc_thresholds(jnp.concatenate([p0, p1]))
    preds = _pack_preds(p0, p1)
    comp, mask = _expand_masks(p0, p1, thr)
    return (preds, comp, mask)


# speedup vs baseline: 2.7023x; 2.0231x over previous
import jax
import jax.numpy as jnp
from jax import lax
from jax.experimental import pallas as pl
from jax.experimental.pallas import tpu as pltpu
from jax.experimental.pallas import tpu_sc as plsc

N = 262144
F = 256
C = 2
K = int(0.1 * N)

NC = 2
NS = 16
L = 16

SC_VALS = N // NS
SC_VREGS = SC_VALS // L

_TOPBIT = -2147483648
_LOWMASK = 0x7FFFFFFF

_MM_BLK = 4096
_PBLK = 8192


def _mm_body(x_ref, wt_ref, b_ref, p0_ref, p1_ref):
    y = (
        jnp.dot(x_ref[...], wt_ref[...], preferred_element_type=jnp.float32)
        + b_ref[0, :]
    )
    p0_ref[...] = y[:, 0]
    p1_ref[...] = y[:, 1]


def _matmul_split(x, W, b):
    wt = W.T
    b2 = b.reshape(1, C)
    return pl.pallas_call(
        _mm_body,
        grid=(N // _MM_BLK,),
        in_specs=[
            pl.BlockSpec((_MM_BLK, F), lambda i: (i, 0)),
            pl.BlockSpec((F, C), lambda i: (0, 0)),
            pl.BlockSpec((1, C), lambda i: (0, 0)),
        ],
        out_specs=[
            pl.BlockSpec((_MM_BLK,), lambda i: (i,)),
            pl.BlockSpec((_MM_BLK,), lambda i: (i,)),
        ],
        out_shape=[
            jax.ShapeDtypeStruct((N,), jnp.float32),
            jax.ShapeDtypeStruct((N,), jnp.float32),
        ],
    )(x, wt, b2)


def _sc_body(pcat_hbm, thr_hbm, data_v, keys_v, hist_v, tot_v, tmp_v,
             acc_v, shared_tot):
    cid = lax.axis_index("c")
    sid = lax.axis_index("s")

    lane = lax.iota(jnp.int32, L)
    ones_i = jnp.ones((L,), jnp.int32)
    zeros_i = jnp.zeros((L,), jnp.int32)

    base = cid * N + sid * SC_VALS
    pltpu.sync_copy(pcat_hbm.at[pl.ds(base, SC_VALS)], data_v)

    def _mk_keys(j, _):
        v = data_v[pl.ds(j * L, L)]
        bits = plsc.bitcast(v, jnp.int32)
        m = jnp.right_shift(bits, 31)
        key = bits ^ (m | _TOPBIT)
        keys_v[pl.ds(j * L, L)] = plsc.bitcast(key, jnp.uint32)
        return 0

    lax.fori_loop(0, SC_VREGS, _mk_keys, 0)

    prefix = jnp.uint32(0)
    rem = jnp.int32(K)

    for p in range(4):
        shift = 24 - 8 * p

        def _zero_hist(j, _):
            hist_v[pl.ds(j * L, L)] = zeros_i
            return 0

        lax.fori_loop(0, 256, _zero_hist, 0)

        pfx_vec = jnp.full((L,), prefix, jnp.uint32)

        def _hist_pass(j, _):
            key = keys_v[pl.ds(j * L, L)]
            binv = lax.convert_element_type(
                lax.shift_right_logical(key, jnp.uint32(shift)) & jnp.uint32(255),
                jnp.int32)
            idx = binv * L + lane
            if p == 0:
                plsc.addupdate_scatter(hist_v, [idx], ones_i)
            else:
                hi = lax.shift_right_logical(key, jnp.uint32(shift + 8))
                plsc.addupdate_scatter(hist_v, [idx], ones_i, mask=hi == pfx_vec)
            return 0

        lax.fori_loop(0, SC_VREGS, _hist_pass, 0)

        def _reduce(g, _):
            bstart = g * L
            row_base = (bstart + lane) * L
            acc = zeros_i
            for l in range(L):
                acc = acc + plsc.load_gather(hist_v, [row_base + l])
            tot_v[pl.ds(bstart, L)] = acc
            return 0

        lax.fori_loop(0, 16, _reduce, 0)

        pltpu.sync_copy(tot_v, shared_tot.at[pl.ds(sid * 256, 256)])
        plsc.subcore_barrier()
        pltpu.sync_copy(shared_tot, tmp_v)
        plsc.subcore_barrier()

        for i in range(16):
            acc_v[pl.ds(i * L, L)] = zeros_i

        def _combine(t, _):
            for i in range(16):
                acc_v[pl.ds(i * L, L)] = (
                    acc_v[pl.ds(i * L, L)] + tmp_v[pl.ds(t * 256 + i * L, L)]
                )
            return 0

        lax.fori_loop(0, NS, _combine, 0)

        chunks = [acc_v[pl.ds(i * L, L)] for i in range(16)]
        csum = [jnp.sum(ch) for ch in chunks]
        S = [jnp.int32(0)] * 17
        for i in range(15, -1, -1):
            S[i] = S[i + 1] + csum[i]
        nbins = jnp.int32(0)
        for i in range(16):
            rch = lax.rev(chunks[i], (0,))
            sfx_rev = plsc.cumsum(rch) + S[i + 1]
            nbins = nbins + jnp.sum(
                jnp.where(sfx_rev >= rem, ones_i, zeros_i))
        bstar = nbins - 1
        above = jnp.int32(0)
        for i in range(16):
            bin_ids = jnp.int32(i * L) + lane
            above = above + jnp.sum(
                jnp.where(bin_ids > bstar, chunks[i], zeros_i))
        rem = rem - above
        prefix = (prefix << jnp.uint32(8)) | lax.convert_element_type(
            bstar, jnp.uint32)

    sthr = plsc.bitcast(jnp.full((L,), prefix ^ jnp.uint32(0x80000000),
                                 jnp.uint32), jnp.int32)
    tot_v[pl.ds(0, L)] = sthr
    slot = (cid * NS + sid) * L
    pltpu.sync_copy(tot_v.at[pl.ds(0, L)], thr_hbm.at[pl.ds(slot, L)])


@jax.jit
def _sc_thresholds(pcat):
    mesh = plsc.VectorSubcoreMesh(
        core_axis_name="c", subcore_axis_name="s", num_cores=NC, num_subcores=NS)
    return pl.kernel(
        _sc_body,
        out_type=jax.ShapeDtypeStruct((NC * NS * L,), jnp.int32),
        mesh=mesh,
        compiler_params=pltpu.CompilerParams(needs_layout_passes=False),
        scratch_types=[
            pltpu.VMEM((SC_VALS,), jnp.float32),
            pltpu.VMEM((SC_VALS,), jnp.uint32),
            pltpu.VMEM((256 * L,), jnp.int32),
            pltpu.VMEM((256,), jnp.int32),
            pltpu.VMEM((NS * 256,), jnp.int32),
            pltpu.VMEM((256,), jnp.int32),
            pltpu.VMEM_SHARED((NS * 256,), jnp.int32),
        ],
    )(pcat)


def _rowsel_body(thr_ref, p0_ref, p1_ref, o_ref):
    t0 = thr_ref[0]
    t1 = thr_ref[NS * L]

    def skey(v):
        bits = v.view(jnp.int32)
        m = jnp.right_shift(bits, 31)
        return bits ^ (m & _LOWMASK)

    sel = (skey(p0_ref[...]) >= t0) | (skey(p1_ref[...]) >= t1)
    o_ref[...] = jnp.where(sel, jnp.float32(1.0), jnp.float32(0.0))


def _rowsel(p0, p1, thr):
    return pl.pallas_call(
        _rowsel_body,
        grid=(N // _PBLK,),
        in_specs=[
            pl.BlockSpec(memory_space=pltpu.SMEM),
            pl.BlockSpec((_PBLK,), lambda i: (i,)),
            pl.BlockSpec((_PBLK,), lambda i: (i,)),
        ],
        out_specs=pl.BlockSpec((_PBLK,), lambda i: (i,)),
        out_shape=jax.ShapeDtypeStruct((N,), jnp.float32),
    )(thr, p0, p1)


def kernel(x, W, b, single_label):
    p0, p1 = _matmul_split(x, W, b)
    thr = _sc_thresholds(jnp.concatenate([p0, p1]))
    row = _rowsel(p0, p1, thr)
    preds = jnp.stack([p0, p1], axis=1)
    comp = jnp.stack([row, row], axis=1)
    mask = jnp.stack([row, row], axis=1)
    return (preds, comp, mask)


# speedup vs baseline: 3.5231x; 1.3037x over previous
import jax
import jax.numpy as jnp
from jax import lax
from jax.experimental import pallas as pl
from jax.experimental.pallas import tpu as pltpu
from jax.experimental.pallas import tpu_sc as plsc

N = 262144
F = 256
C = 2
K = int(0.1 * N)

NC = 2
NS = 16
L = 16

SC_VALS = N // NS
SC_VREGS = SC_VALS // L

_TOPBIT = -2147483648
_LOWMASK = 0x7FFFFFFF

_MM_BLK = 4096
_PBLK = 8192


def _mm_body(x_ref, w_ref, b_ref, p0_ref, p1_ref):
    yt = lax.dot_general(
        w_ref[...], x_ref[...], (((1,), (1,)), ((), ())),
        preferred_element_type=jnp.float32)
    p0_ref[...] = yt[0, :] + b_ref[0]
    p1_ref[...] = yt[1, :] + b_ref[1]


def _matmul_split(x, W, b):
    return pl.pallas_call(
        _mm_body,
        grid=(N // _MM_BLK,),
        in_specs=[
            pl.BlockSpec((_MM_BLK, F), lambda i: (i, 0)),
            pl.BlockSpec((C, F), lambda i: (0, 0)),
            pl.BlockSpec(memory_space=pltpu.SMEM),
        ],
        out_specs=[
            pl.BlockSpec((_MM_BLK,), lambda i: (i,)),
            pl.BlockSpec((_MM_BLK,), lambda i: (i,)),
        ],
        out_shape=[
            jax.ShapeDtypeStruct((N,), jnp.float32),
            jax.ShapeDtypeStruct((N,), jnp.float32),
        ],
    )(x, W, b)


def _sc_body(pcat_hbm, thr_hbm, data_v, keys_v, hist_v, tot_v, tmp_v,
             acc_v, shared_tot):
    cid = lax.axis_index("c")
    sid = lax.axis_index("s")

    lane = lax.iota(jnp.int32, L)
    ones_i = jnp.ones((L,), jnp.int32)
    zeros_i = jnp.zeros((L,), jnp.int32)

    base = cid * N + sid * SC_VALS
    pltpu.sync_copy(pcat_hbm.at[pl.ds(base, SC_VALS)], data_v)

    def _mk_keys(j, _):
        v = data_v[pl.ds(j * L, L)]
        bits = plsc.bitcast(v, jnp.int32)
        m = jnp.right_shift(bits, 31)
        key = bits ^ (m | _TOPBIT)
        keys_v[pl.ds(j * L, L)] = plsc.bitcast(key, jnp.uint32)
        return 0

    lax.fori_loop(0, SC_VREGS, _mk_keys, 0)

    prefix = jnp.uint32(0)
    rem = jnp.int32(K)

    for p in range(4):
        shift = 24 - 8 * p

        def _zero_hist(j, _):
            hist_v[pl.ds(j * L, L)] = zeros_i
            return 0

        lax.fori_loop(0, 256, _zero_hist, 0)

        pfx_vec = jnp.full((L,), prefix, jnp.uint32)

        def _hist_pass(j, _):
            key = keys_v[pl.ds(j * L, L)]
            binv = lax.convert_element_type(
                lax.shift_right_logical(key, jnp.uint32(shift)) & jnp.uint32(255),
                jnp.int32)
            idx = binv * L + lane
            if p == 0:
                plsc.addupdate_scatter(hist_v, [idx], ones_i)
            else:
                hi = lax.shift_right_logical(key, jnp.uint32(shift + 8))
                plsc.addupdate_scatter(hist_v, [idx], ones_i, mask=hi == pfx_vec)
            return 0

        lax.fori_loop(0, SC_VREGS, _hist_pass, 0)

        def _reduce(g, _):
            bstart = g * L
            row_base = (bstart + lane) * L
            acc = zeros_i
            for l in range(L):
                acc = acc + plsc.load_gather(hist_v, [row_base + l])
            tot_v[pl.ds(bstart, L)] = acc
            return 0

        lax.fori_loop(0, 16, _reduce, 0)

        pltpu.sync_copy(tot_v, shared_tot.at[pl.ds(sid * 256, 256)])
        plsc.subcore_barrier()
        pltpu.sync_copy(shared_tot, tmp_v)
        plsc.subcore_barrier()

        for i in range(16):
            acc_v[pl.ds(i * L, L)] = zeros_i

        def _combine(t, _):
            for i in range(16):
                acc_v[pl.ds(i * L, L)] = (
                    acc_v[pl.ds(i * L, L)] + tmp_v[pl.ds(t * 256 + i * L, L)]
                )
            return 0

        lax.fori_loop(0, NS, _combine, 0)

        chunks = [acc_v[pl.ds(i * L, L)] for i in range(16)]
        csum = [jnp.sum(ch) for ch in chunks]
        S = [jnp.int32(0)] * 17
        for i in range(15, -1, -1):
            S[i] = S[i + 1] + csum[i]
        nbins = jnp.int32(0)
        for i in range(16):
            rch = lax.rev(chunks[i], (0,))
            sfx_rev = plsc.cumsum(rch) + S[i + 1]
            nbins = nbins + jnp.sum(
                jnp.where(sfx_rev >= rem, ones_i, zeros_i))
        bstar = nbins - 1
        above = jnp.int32(0)
        for i in range(16):
            bin_ids = jnp.int32(i * L) + lane
            above = above + jnp.sum(
                jnp.where(bin_ids > bstar, chunks[i], zeros_i))
        rem = rem - above
        prefix = (prefix << jnp.uint32(8)) | lax.convert_element_type(
            bstar, jnp.uint32)

    sthr = plsc.bitcast(jnp.full((L,), prefix ^ jnp.uint32(0x80000000),
                                 jnp.uint32), jnp.int32)
    tot_v[pl.ds(0, L)] = sthr
    slot = (cid * NS + sid) * L
    pltpu.sync_copy(tot_v.at[pl.ds(0, L)], thr_hbm.at[pl.ds(slot, L)])


@jax.jit
def _sc_thresholds(pcat):
    mesh = plsc.VectorSubcoreMesh(
        core_axis_name="c", subcore_axis_name="s", num_cores=NC, num_subcores=NS)
    return pl.kernel(
        _sc_body,
        out_type=jax.ShapeDtypeStruct((NC * NS * L,), jnp.int32),
        mesh=mesh,
        compiler_params=pltpu.CompilerParams(needs_layout_passes=False),
        scratch_types=[
            pltpu.VMEM((SC_VALS,), jnp.float32),
            pltpu.VMEM((SC_VALS,), jnp.uint32),
            pltpu.VMEM((256 * L,), jnp.int32),
            pltpu.VMEM((256,), jnp.int32),
            pltpu.VMEM((NS * 256,), jnp.int32),
            pltpu.VMEM((256,), jnp.int32),
            pltpu.VMEM_SHARED((NS * 256,), jnp.int32),
        ],
    )(pcat)


def _rowsel_body(thr_ref, p0_ref, p1_ref, o_ref):
    t0 = thr_ref[0]
    t1 = thr_ref[NS * L]

    def skey(v):
        bits = v.view(jnp.int32)
        m = jnp.right_shift(bits, 31)
        return bits ^ (m & _LOWMASK)

    sel = (skey(p0_ref[...]) >= t0) | (skey(p1_ref[...]) >= t1)
    o_ref[...] = jnp.where(sel, jnp.float32(1.0), jnp.float32(0.0))


def _rowsel(p0, p1, thr):
    return pl.pallas_call(
        _rowsel_body,
        grid=(N // _PBLK,),
        in_specs=[
            pl.BlockSpec(memory_space=pltpu.SMEM),
            pl.BlockSpec((_PBLK,), lambda i: (i,)),
            pl.BlockSpec((_PBLK,), lambda i: (i,)),
        ],
        out_specs=pl.BlockSpec((_PBLK,), lambda i: (i,)),
        out_shape=jax.ShapeDtypeStruct((N,), jnp.float32),
    )(thr, p0, p1)


def kernel(x, W, b, single_label):
    p0, p1 = _matmul_split(x, W, b)
    thr = _sc_thresholds(jnp.concatenate([p0, p1]))
    row = _rowsel(p0, p1, thr)
    preds = jnp.stack([p0, p1], axis=1)
    comp = jnp.stack([row, row], axis=1)
    mask = jnp.stack([row, row], axis=1)
    return (preds, comp, mask)


# speedup vs baseline: 3.8379x; 1.0894x over previous
import jax
import jax.numpy as jnp
from jax import lax
from jax.experimental import pallas as pl
from jax.experimental.pallas import tpu as pltpu
from jax.experimental.pallas import tpu_sc as plsc

N = 262144
F = 256
C = 2
K = int(0.1 * N)

NC = 2
NS = 16
L = 16

SC_VALS = N // NS
SC_VREGS = SC_VALS // L

_TOPBIT = -2147483648
_LOWMASK = 0x7FFFFFFF

_MM_BLK = 8192
_PBLK = 8192


def _mm_body(x_ref, w_ref, b_ref, p0_ref, p1_ref):
    yt = lax.dot_general(
        w_ref[...], x_ref[...], (((1,), (1,)), ((), ())),
        preferred_element_type=jnp.float32)
    p0_ref[...] = yt[0, :] + b_ref[0]
    p1_ref[...] = yt[1, :] + b_ref[1]


def _matmul_split(x, W, b):
    return pl.pallas_call(
        _mm_body,
        grid=(N // _MM_BLK,),
        in_specs=[
            pl.BlockSpec((_MM_BLK, F), lambda i: (i, 0)),
            pl.BlockSpec((C, F), lambda i: (0, 0)),
            pl.BlockSpec(memory_space=pltpu.SMEM),
        ],
        out_specs=[
            pl.BlockSpec((_MM_BLK,), lambda i: (i,)),
            pl.BlockSpec((_MM_BLK,), lambda i: (i,)),
        ],
        out_shape=[
            jax.ShapeDtypeStruct((N,), jnp.float32),
            jax.ShapeDtypeStruct((N,), jnp.float32),
        ],
    )(x, W, b)


def _sc_body(pcat_hbm, thr_hbm, data_v, keys_v, hist_v, tot_v, tmp_v,
             acc_v, shared_tot):
    cid = lax.axis_index("c")
    sid = lax.axis_index("s")

    lane = lax.iota(jnp.int32, L)
    ones_i = jnp.ones((L,), jnp.int32)
    zeros_i = jnp.zeros((L,), jnp.int32)

    base = cid * N + sid * SC_VALS
    pltpu.sync_copy(pcat_hbm.at[pl.ds(base, SC_VALS)], data_v)

    def _mk_keys(j, _):
        v = data_v[pl.ds(j * L, L)]
        bits = plsc.bitcast(v, jnp.int32)
        m = jnp.right_shift(bits, 31)
        key = bits ^ (m | _TOPBIT)
        keys_v[pl.ds(j * L, L)] = plsc.bitcast(key, jnp.uint32)
        return 0

    lax.fori_loop(0, SC_VREGS, _mk_keys, 0)

    prefix = jnp.uint32(0)
    rem = jnp.int32(K)

    for p in range(4):
        shift = 24 - 8 * p

        def _zero_hist(j, _):
            hist_v[pl.ds(j * L, L)] = zeros_i
            return 0

        lax.fori_loop(0, 256, _zero_hist, 0)

        pfx_vec = jnp.full((L,), prefix, jnp.uint32)

        def _hist_pass(j, _):
            key = keys_v[pl.ds(j * L, L)]
            binv = lax.convert_element_type(
                lax.shift_right_logical(key, jnp.uint32(shift)) & jnp.uint32(255),
                jnp.int32)
            idx = binv * L + lane
            if p == 0:
                plsc.addupdate_scatter(hist_v, [idx], ones_i)
            else:
                hi = lax.shift_right_logical(key, jnp.uint32(shift + 8))
                plsc.addupdate_scatter(hist_v, [idx], ones_i, mask=hi == pfx_vec)
            return 0

        lax.fori_loop(0, SC_VREGS, _hist_pass, 0)

        def _reduce(g, _):
            bstart = g * L
            row_base = (bstart + lane) * L
            acc = zeros_i
            for l in range(L):
                acc = acc + plsc.load_gather(hist_v, [row_base + l])
            tot_v[pl.ds(bstart, L)] = acc
            return 0

        lax.fori_loop(0, 16, _reduce, 0)

        pltpu.sync_copy(tot_v, shared_tot.at[pl.ds(sid * 256, 256)])
        plsc.subcore_barrier()
        pltpu.sync_copy(shared_tot, tmp_v)
        plsc.subcore_barrier()

        for i in range(16):
            acc_v[pl.ds(i * L, L)] = zeros_i

        def _combine(t, _):
            for i in range(16):
                acc_v[pl.ds(i * L, L)] = (
                    acc_v[pl.ds(i * L, L)] + tmp_v[pl.ds(t * 256 + i * L, L)]
                )
            return 0

        lax.fori_loop(0, NS, _combine, 0)

        chunks = [acc_v[pl.ds(i * L, L)] for i in range(16)]
        csum = [jnp.sum(ch) for ch in chunks]
        S = [jnp.int32(0)] * 17
        for i in range(15, -1, -1):
            S[i] = S[i + 1] + csum[i]
        nbins = jnp.int32(0)
        for i in range(16):
            rch = lax.rev(chunks[i], (0,))
            sfx_rev = plsc.cumsum(rch) + S[i + 1]
            nbins = nbins + jnp.sum(
                jnp.where(sfx_rev >= rem, ones_i, zeros_i))
        bstar = nbins - 1
        above = jnp.int32(0)
        for i in range(16):
            bin_ids = jnp.int32(i * L) + lane
            above = above + jnp.sum(
                jnp.where(bin_ids > bstar, chunks[i], zeros_i))
        rem = rem - above
        prefix = (prefix << jnp.uint32(8)) | lax.convert_element_type(
            bstar, jnp.uint32)

    sthr = plsc.bitcast(jnp.full((L,), prefix ^ jnp.uint32(0x80000000),
                                 jnp.uint32), jnp.int32)
    tot_v[pl.ds(0, L)] = sthr
    slot = (cid * NS + sid) * L
    pltpu.sync_copy(tot_v.at[pl.ds(0, L)], thr_hbm.at[pl.ds(slot, L)])


@jax.jit
def _sc_thresholds(pcat):
    mesh = plsc.VectorSubcoreMesh(
        core_axis_name="c", subcore_axis_name="s", num_cores=NC, num_subcores=NS)
    return pl.kernel(
        _sc_body,
        out_type=jax.ShapeDtypeStruct((NC * NS * L,), jnp.int32),
        mesh=mesh,
        compiler_params=pltpu.CompilerParams(needs_layout_passes=False),
        scratch_types=[
            pltpu.VMEM((SC_VALS,), jnp.float32),
            pltpu.VMEM((SC_VALS,), jnp.uint32),
            pltpu.VMEM((256 * L,), jnp.int32),
            pltpu.VMEM((256,), jnp.int32),
            pltpu.VMEM((NS * 256,), jnp.int32),
            pltpu.VMEM((256,), jnp.int32),
            pltpu.VMEM_SHARED((NS * 256,), jnp.int32),
        ],
    )(pcat)


def _rowsel_body(thr_ref, p0_ref, p1_ref, o_ref):
    t0 = thr_ref[0]
    t1 = thr_ref[NS * L]

    def skey(v):
        bits = v.view(jnp.int32)
        m = jnp.right_shift(bits, 31)
        return bits ^ (m & _LOWMASK)

    sel = (skey(p0_ref[...]) >= t0) | (skey(p1_ref[...]) >= t1)
    o_ref[...] = jnp.where(sel, jnp.float32(1.0), jnp.float32(0.0))


def _rowsel(p0, p1, thr):
    return pl.pallas_call(
        _rowsel_body,
        grid=(N // _PBLK,),
        in_specs=[
            pl.BlockSpec(memory_space=pltpu.SMEM),
            pl.BlockSpec((_PBLK,), lambda i: (i,)),
            pl.BlockSpec((_PBLK,), lambda i: (i,)),
        ],
        out_specs=pl.BlockSpec((_PBLK,), lambda i: (i,)),
        out_shape=jax.ShapeDtypeStruct((N,), jnp.float32),
    )(thr, p0, p1)


def kernel(x, W, b, single_label):
    p0, p1 = _matmul_split(x, W, b)
    thr = _sc_thresholds(jnp.concatenate([p0, p1]))
    row = _rowsel(p0, p1, thr)
    preds = jnp.stack([p0, p1], axis=1)
    comp = jnp.stack([row, row], axis=1)
    mask = jnp.stack([row, row], axis=1)
    return (preds, comp, mask)
